# baseline (speedup 1.0000x reference)
import functools
import math

import jax
import jax.numpy as jnp
from jax import lax
from jax.experimental import pallas as pl
from jax.experimental.pallas import tpu as pltpu
from jax.experimental.pallas import tpu_sc as plsc

_NC = 2
_NS = 16
_NW = _NC * _NS
_L = 16

_INT_MAX = jnp.iinfo(jnp.int32).max
_LN2 = 0.6931471805599453


def _bisect(keys_ref, q, num_keys, steps):
    lo = jnp.zeros((_L,), jnp.int32)
    hi = jnp.full((_L,), num_keys, jnp.int32)
    for _ in range(steps):
        mid = lax.shift_right_arithmetic(lo + hi, 1)
        kv = plsc.load_gather(keys_ref, [mid])
        pred = kv < q
        lo = jnp.where(pred, mid + 1, lo)
        hi = jnp.where(pred, hi, mid)
    idxc = jnp.minimum(lo, num_keys - 1)
    kv = plsc.load_gather(keys_ref, [idxc])
    return idxc, kv == q


def _vlog(x):
    bits = plsc.bitcast(x, jnp.int32)
    e = (lax.shift_right_logical(bits, 23) & 0xFF) - 127
    m = plsc.bitcast((bits & 0x7FFFFF) | 0x3F800000, jnp.float32)
    big = m > (4.0 / 3.0)
    m = jnp.where(big, m * 0.5, m)
    e = jnp.where(big, e + 1, e)
    r = m - 1.0
    s = jnp.float32(-1.0 / 14.0)
    for kk in range(13, 0, -1):
        c = jnp.float32((1.0 if kk % 2 else -1.0) / kk)
        s = s * r + c
    s = s * r
    return e.astype(jnp.float32) * jnp.float32(_LN2) + s


_GC = 16


def _make_sc_stage(n, seq_len, vocab, c1, c2):
    chunk = n // _NW
    steps1 = max(1, math.ceil(math.log2(c1 + 1)))
    steps2 = max(1, math.ceil(math.log2(c2 + 1)))
    nt = (vocab + 127) // 128
    vp128 = nt * 128
    ntm = vocab // 128
    nvec = vp128 // _L
    nch = chunk // _GC
    mesh = plsc.VectorSubcoreMesh(
        core_axis_name="c", subcore_axis_name="s",
        num_cores=_NC, num_subcores=_NS)

    @functools.partial(
        pl.kernel,
        out_type=jax.ShapeDtypeStruct((n * nt, 128), jnp.float32),
        mesh=mesh,
        compiler_params=pltpu.CompilerParams(needs_layout_passes=False),
        scratch_types=[
            pltpu.VMEM((chunk + 8,), jnp.int32),
            pltpu.VMEM((c1 + _L,), jnp.int32),
            pltpu.VMEM((c2 + _L,), jnp.int32),
            pltpu.VMEM((chunk,), jnp.int32),
            pltpu.VMEM((chunk,), jnp.int32),
            pltpu.VMEM((chunk * nt,), jnp.int32),
            pltpu.VMEM((_GC * nt, 128), jnp.float32),
            pltpu.VMEM((_GC * nt, 128), jnp.float32),
            pltpu.VMEM((vp128,), jnp.float32),
            pltpu.VMEM((nt * 8, 128), jnp.float32),
            pltpu.VMEM((nt * 8, 128), jnp.float32),
            pltpu.VMEM((nt, 128), jnp.float32),
            pltpu.VMEM((_L,), jnp.float32),
            pltpu.SemaphoreType.DMA,
            pltpu.SemaphoreType.DMA,
            pltpu.SemaphoreType.DMA,
            pltpu.SemaphoreType.DMA,
            pltpu.SemaphoreType.DMA,
        ],
    )
    def sc_stage(batch_hbm, k1_hbm, k2_hbm, t8_hbm, p0_hbm, p1_hbm, p2_hbm,
                 p1t_hbm, p2t_hbm, al_hbm, out_hbm,
                 qbuf, k1v, k2v, selv, encv, idxall, gbufa, gbufb, p0v,
                 tb1, tb2, rowo, alv, gs0, gs1, ws0, ws1, fsem):
        wid = lax.axis_index("s") * _NC + lax.axis_index("c")
        base = wid * chunk
        pltpu.sync_copy(k1_hbm, k1v.at[pl.ds(0, c1)])
        pltpu.sync_copy(k2_hbm, k2v.at[pl.ds(0, c2)])
        k1v[pl.ds(c1, _L)] = jnp.full((_L,), _INT_MAX, jnp.int32)
        k2v[pl.ds(c2, _L)] = jnp.full((_L,), _INT_MAX, jnp.int32)
        pltpu.sync_copy(al_hbm, alv)
        pltpu.sync_copy(p0_hbm, p0v.at[pl.ds(0, vocab)])
        pltpu.sync_copy(batch_hbm.at[pl.ds(base, chunk)],
                        qbuf.at[pl.ds(8, chunk)])
        @pl.when(wid > 0)
        def _():
            pltpu.sync_copy(batch_hbm.at[pl.ds(base - 8, 8)],
                            qbuf.at[pl.ds(0, 8)])

        iota = lax.iota(jnp.int32, _L)
        c1vec = jnp.full((_L,), c1, jnp.int32)

        def lookup_step(t, carry):
            off = t * _L
            j = lax.rem(base + off, seq_len) + iota
            idxs = off + 8 + iota
            q1 = plsc.load_gather(qbuf, [idxs - 1])
            t2 = plsc.load_gather(qbuf, [idxs - 2])
            i1, hit1 = _bisect(k1v, q1, c1, steps1)
            q2 = t2 * vocab + q1
            i2, hit2 = _bisect(k2v, q2, c2, steps2)
            f1 = hit1 & (j >= 1)
            f2 = hit2 & (j >= 2)
            selv[pl.ds(off, _L)] = jnp.where(f1 & ~f2, i1, c1vec)
            i1p = jnp.where(f1, i1 + 1, 0)
            enc = (lax.shift_left(i2, 12) | lax.shift_left(i1p, 1)
                   | jnp.where(f2, 1, 0))
            encv[pl.ds(off, _L)] = jnp.where(f2, enc, 0)
            lane_sub = iota & (nt - 1)
            lane_pos = lax.shift_right_logical(iota, 3)
            for g in range(nt):
                sv = plsc.load_gather(selv, [off + g * 2 + lane_pos])
                idxall[pl.ds(off * nt + g * _L, _L)] = sv * nt + lane_sub
            return carry

        lax.fori_loop(0, chunk // _L, lookup_step, 0)

        bufs = (gbufa, gbufb)
        gsems = (gs0, gs1)
        wsems = (ws0, ws1)
        rows_per = _GC * nt

        def start_gather(c, b):
            return pltpu.async_copy(
                t8_hbm.at[idxall.at[pl.ds(c * rows_per, rows_per)]],
                bufs[b], gsems[b])

        gh = [None] * nch
        wh = [None] * nch
        gh[0] = start_gather(0, 0)
        for c in range(nch):
            gh[c].wait()
            if c + 1 < nch:
                if c - 1 >= 0:
                    wh[c - 1].wait()
                gh[c + 1] = start_gather(c + 1, (c + 1) % 2)
            wh[c] = pltpu.async_copy(
                bufs[c % 2],
                out_hbm.at[pl.ds(pl.multiple_of((base + c * _GC) * nt, 8),
                                 rows_per)],
                wsems[c % 2])
        if nch >= 2:
            wh[nch - 2].wait()
        wh[nch - 1].wait()

        av = alv[...]
        fzero = jnp.float32(0.0)
        a0 = jnp.max(jnp.where(iota == 0, av, fzero))
        a1 = jnp.max(jnp.where(iota == 1, av, fzero))
        a2 = jnp.max(jnp.where(iota == 2, av, fzero))
        uni = jnp.float32(1.0 / vocab)
        izero = jnp.zeros((_L,), jnp.int32)

        def fetch_band(src, tail_src, band, dst):
            band = pl.multiple_of(band, 8)
            hs = []
            for ct in range(ntm):
                hs.append(pltpu.async_copy(
                    src.at[pl.ds(band, 8), pl.ds(ct * 128, 128)],
                    dst.at[pl.ds(ct * 8, 8)], fsem))
            if ntm < nt:
                hs.append(pltpu.async_copy(
                    tail_src.at[pl.ds(band, 8)],
                    dst.at[pl.ds(ntm * 8, 8)], fsem))
            return hs

        def fix_group(g, carry):
            encg = encv[pl.ds(g * _L, _L)]

            @pl.when(jnp.max(encg) != 0)
            def _():
                def fix_lane(lane, carry2):
                    enc = jnp.max(jnp.where(iota == lane, encg, izero))

                    @pl.when(enc != 0)
                    def _():
                        i1p = lax.shift_right_logical(enc, 1) & 0x7FF
                        i2 = lax.shift_right_logical(enc, 12)
                        i1idx = jnp.maximum(i1p - 1, 0)
                        sub1 = lax.rem(i1idx, 8)
                        sub2 = lax.rem(i2, 8)
                        hs = fetch_band(p1_hbm, p1t_hbm, i1idx - sub1, tb1)
                        hs += fetch_band(p2_hbm, p2t_hbm, i2 - sub2, tb2)
                        for h in hs:
                            h.wait()
                        c1c = jnp.where(i1p > 0, a1, fzero)
                        missc = (a1 - c1c) * uni

                        def vec_step(v, carry3):
                            ct = lax.shift_right_logical(v, 3)
                            part = pl.ds((v & 7) * _L, _L)
                            r1 = tb1[ct * 8 + sub1, part]
                            r2 = tb2[ct * 8 + sub2, part]
                            x = (a0 * p0v[pl.ds(v * _L, _L)] + missc
                                 + c1c * r1 + a2 * r2)
                            rowo[ct, part] = _vlog(x)
                            return carry3

                        lax.fori_loop(0, nvec, vec_step, 0)
                        orow = pl.multiple_of(
                            (base + g * _L + lane) * nt, 8)
                        pltpu.sync_copy(rowo,
                                        out_hbm.at[pl.ds(orow, nt)])

                    return carry2

                lax.fori_loop(0, _L, fix_lane, 0)

            return carry

        lax.fori_loop(0, chunk // _L, fix_group, 0)

    return sc_stage


def _tail_body(in_ref, out_ref):
    out_ref[...] = in_ref[...]


def _make_tail_copy(rows, vocab):
    cb = vocab // 128
    br = 2000 if rows % 2000 == 0 else (rows if rows <= 4096 else 8)
    grid_spec = pl.GridSpec(
        grid=(rows // br,),
        in_specs=[pl.BlockSpec((br, 128), lambda i: (i, cb))],
        out_specs=pl.BlockSpec((br, 128), lambda i: (i, 0)),
    )
    return pl.pallas_call(
        _tail_body,
        grid_spec=grid_spec,
        out_shape=jax.ShapeDtypeStruct((rows, 128), jnp.float32),
        compiler_params=pltpu.CompilerParams(
            dimension_semantics=("arbitrary",)),
    )


_TR = 8


def _table_body(p0_ref, al_ref, rows_ref, out_ref, *, nsteps, uni):
    i = pl.program_id(0)
    a0 = al_ref[0]
    a1 = al_ref[1]
    a2 = al_ref[2]
    vp = p0_ref.shape[1]
    base = a0 * p0_ref[0, :] + a2 * uni

    @pl.when(i < nsteps - 1)
    def _():
        vals = jnp.log(base[None, :] + a1 * rows_ref[...])
        out_ref[...] = vals.reshape(_TR, vp // 128, 128).reshape(-1, 128)

    @pl.when(i == nsteps - 1)
    def _():
        row = jnp.log(base + a1 * uni)
        vals = jnp.broadcast_to(row[None, :], (_TR, vp))
        out_ref[...] = vals.reshape(_TR, vp // 128, 128).reshape(-1, 128)


def _make_tc_table(vp, c1, true_vocab):
    nt = vp // 128
    nblk = c1 // _TR
    nsteps = nblk + 1
    grid_spec = pl.GridSpec(
        grid=(nsteps,),
        in_specs=[
            pl.BlockSpec((1, vp), lambda i: (0, 0)),
            pl.BlockSpec(memory_space=pltpu.SMEM),
            pl.BlockSpec((_TR, vp), lambda i: (jnp.minimum(i, nblk - 1), 0)),
        ],
        out_specs=pl.BlockSpec((_TR * nt, 128), lambda i: (i, 0)),
    )
    return pl.pallas_call(
        functools.partial(_table_body, nsteps=nsteps, uni=1.0 / true_vocab),
        grid_spec=grid_spec,
        out_shape=jax.ShapeDtypeStruct(((c1 + _TR) * nt, 128), jnp.float32),
        compiler_params=pltpu.CompilerParams(
            dimension_semantics=("arbitrary",)),
    )


def kernel(batch, alphas, p0, ctx1_keys, ctx1_probs, ctx2_keys, ctx2_probs):
    b, s = batch.shape
    vocab = p0.shape[0]
    c1 = ctx1_keys.shape[0]
    c2 = ctx2_keys.shape[0]
    n = b * s
    flat = batch.reshape(n).astype(jnp.int32)
    al16 = jnp.concatenate([alphas, jnp.zeros((13,), jnp.float32)])

    nt = (vocab + 127) // 128
    vp128 = nt * 128
    padc = vp128 - vocab
    p0p = jnp.pad(p0.reshape(1, vocab), ((0, 0), (0, padc)),
                  constant_values=1.0)
    ctx1p = jnp.pad(ctx1_probs, ((0, 0), (0, padc)), constant_values=1.0)
    table8 = _make_tc_table(vp128, c1, vocab)(p0p, alphas, ctx1p)

    if vocab % 128:
        p1t = _make_tail_copy(c1, vocab)(ctx1_probs)
        p2t = _make_tail_copy(c2, vocab)(ctx2_probs)
    else:
        p1t = jnp.zeros((c1, 128), jnp.float32)
        p2t = jnp.zeros((c2, 128), jnp.float32)

    sc_stage = _make_sc_stage(n, s, vocab, c1, c2)
    out8 = sc_stage(flat, ctx1_keys, ctx2_keys, table8, p0,
                    ctx1_probs, ctx2_probs, p1t, p2t, al16)
    return out8.reshape(n, vp128)[:, :vocab].reshape(b, s, vocab)
